# Initial kernel scaffold; baseline (speedup 1.0000x reference)
import jax
import jax.numpy as jnp
from jax.experimental import pallas as pl


def kernel(x, edge_index, edge_attr, batch, loop_edge, loop_pair, metal_idx, node_emb, c1_edge_emb, c1_nn_w, c1_nn_b, c1_root_w, c1_bias, c2_edge_emb, c2_nn_w, c2_nn_b, c2_root_w, c2_bias, c3_edge_emb, c3_nn_w, c3_nn_b, c3_root_w, c3_bias, c4_edge_emb, c4_nn_w, c4_nn_b, c4_root_w, c4_bias, f_w, f_b):
    raise NotImplementedError("write your pallas kernel here")


# speedup vs baseline: 3.4806x; 3.4806x over previous
import functools

import jax
import jax.numpy as jnp
from jax import lax
from jax.experimental import pallas as pl
from jax.experimental.pallas import tpu as pltpu
from jax.experimental.pallas import tpu_sc as plsc

N = 10000
E = 160000
H = 16
B = 8
NPER = N // B
NC = 2
NS = 16
HALF_N = N // NC
HALF_ROWS = HALF_N * 16
G_ROWS = HALF_ROWS + 128
ZPT = G_ROWS // NS
CNT_ROWS = HALF_N + 120
CH = 80
EPT = E // NS
NCHUNK = EPT // CH

_mesh = plsc.VectorSubcoreMesh(
    core_axis_name="c", subcore_axis_name="s", num_cores=NC, num_subcores=NS)


def _scatter_body(with_cnt, h_hbm, src_hbm, dst_hbm, attr_hbm, zeros_hbm,
                  ones_hbm, g_hbm, cnt_hbm, src_v, dst_v, attr_v, key_v,
                  dkey_v, row_v, ones_v, g_sp, cnt_sp, sem):
  c = lax.axis_index("c")
  s = lax.axis_index("s")

  pltpu.sync_copy(zeros_hbm, g_sp.at[pl.ds(s * ZPT, ZPT)])
  if with_cnt:
    cz = CNT_ROWS // NS
    pltpu.sync_copy(zeros_hbm.at[pl.ds(0, cz)],
                    cnt_sp.at[pl.ds(s * cz, cz)])
    pltpu.sync_copy(ones_hbm, ones_v)
  plsc.subcore_barrier()

  def chunk(j, carry):
    off = pl.multiple_of(s * EPT + j * CH, 8)
    pltpu.sync_copy(src_hbm.at[pl.ds(off, CH)], src_v)
    pltpu.sync_copy(dst_hbm.at[pl.ds(off, CH)], dst_v)
    pltpu.sync_copy(attr_hbm.at[pl.ds(off, CH)], attr_v)
    for i in range(CH // 16):
      sl = pl.ds(i * 16, 16)
      d = dst_v[sl]
      a = attr_v[sl]
      k = d * 16 + a - c * HALF_ROWS
      ok = (k >= 0) & (k < HALF_ROWS)
      key_v[sl] = jnp.where(ok, k, HALF_ROWS + (d & 7))
      if with_cnt:
        dl = d - c * HALF_N
        okd = (dl >= 0) & (dl < HALF_N)
        dkey_v[sl] = jnp.where(okd, dl, HALF_N + (d & 7))
    pltpu.async_copy(h_hbm.at[src_v], row_v, sem).wait()
    pltpu.sync_copy(row_v, g_sp.at[key_v], add=True)
    if with_cnt:
      pltpu.sync_copy(ones_v, cnt_sp.at[dkey_v], add=True)
    return carry

  lax.fori_loop(0, NCHUNK, chunk, 0)
  plsc.subcore_barrier()

  dpt = HALF_ROWS // NS
  pltpu.sync_copy(g_sp.at[pl.ds(s * dpt, dpt)],
                  g_hbm.at[pl.ds(c * HALF_ROWS + s * dpt, dpt)])
  if with_cnt:
    @pl.when(s == 0)
    def _():
      pltpu.sync_copy(cnt_sp.at[pl.ds(0, HALF_N)],
                      cnt_hbm.at[pl.ds(c * HALF_N, HALF_N)])


def _make_scatter(with_cnt):
  if with_cnt:
    out_type = [jax.ShapeDtypeStruct((N * 16, H), jnp.float32),
                jax.ShapeDtypeStruct((N, H), jnp.float32)]
  else:
    out_type = jax.ShapeDtypeStruct((N * 16, H), jnp.float32)
  scratch = [
      pltpu.VMEM((CH,), jnp.int32),
      pltpu.VMEM((CH,), jnp.int32),
      pltpu.VMEM((CH,), jnp.int32),
      pltpu.VMEM((CH,), jnp.int32),
      pltpu.VMEM((CH,), jnp.int32),
      pltpu.VMEM((CH, H), jnp.float32),
      pltpu.VMEM((CH, H), jnp.float32),
      pltpu.VMEM_SHARED((G_ROWS, H), jnp.float32),
      pltpu.VMEM_SHARED((CNT_ROWS, H), jnp.float32),
      pltpu.SemaphoreType.DMA,
  ]
  if with_cnt:
    def body(h, src, dst, attr, z, o, g, cnt, *scr):
      _scatter_body(True, h, src, dst, attr, z, o, g, cnt, *scr)
  else:
    def body(h, src, dst, attr, z, o, g, *scr):
      _scatter_body(False, h, src, dst, attr, z, o, g, None, *scr)
  return pl.kernel(body, out_type=out_type, mesh=_mesh,
                   scratch_types=scratch,
                   compiler_params=pltpu.CompilerParams(
                       use_tc_tiling_on_sc=False))


_scatter_cnt = _make_scatter(True)
_scatter = _make_scatter(False)


def _wgen_body(e_ref, w_ref, b_ref, out_ref):
  for l in range(4):
    e = jax.nn.relu(e_ref[l])
    out_ref[l] = jnp.dot(e, w_ref[l],
                         preferred_element_type=jnp.float32, precision=lax.Precision.HIGHEST) + b_ref[l]


def _wgen(estack, wstack, bstack):
  return pl.pallas_call(
      _wgen_body,
      out_shape=jax.ShapeDtypeStruct((4, 16, 256), jnp.float32),
  )(estack, wstack, bstack)


BN0 = 2000


def _h0_body(x_ref, emb_ref, out_ref):
  iota = lax.broadcasted_iota(jnp.int32, (BN0, 128), 1)
  oh = (iota == x_ref[:]).astype(jnp.float32)
  out_ref[:] = jnp.dot(oh, emb_ref[:], preferred_element_type=jnp.float32, precision=lax.Precision.HIGHEST)


def _h0(x2, emb_pad):
  return pl.pallas_call(
      _h0_body,
      grid=(N // BN0,),
      in_specs=[
          pl.BlockSpec((BN0, 1), lambda i: (i, 0)),
          pl.BlockSpec((128, H), lambda i: (0, 0)),
      ],
      out_specs=pl.BlockSpec((BN0, H), lambda i: (i, 0)),
      out_shape=jax.ShapeDtypeStruct((N, H), jnp.float32),
  )(x2, emb_pad)


BN = 2000


def _layer1_body(g_ref, wm_ref, h_ref, rw_ref, b_ref, cnt_ref,
                 out_ref, inv_ref):
  inv = 1.0 / jnp.maximum(cnt_ref[:], 1.0)
  inv_ref[:] = inv
  v = (jnp.dot(g_ref[:], wm_ref[:], preferred_element_type=jnp.float32, precision=lax.Precision.HIGHEST) * inv
       + jnp.dot(h_ref[:], rw_ref[:], preferred_element_type=jnp.float32, precision=lax.Precision.HIGHEST)
       + b_ref[:])
  out_ref[:] = jnp.maximum(v, 0.0)


def _layerk_body(relu, g_ref, wm_ref, h_ref, rw_ref, b_ref, inv_ref, out_ref):
  v = (jnp.dot(g_ref[:], wm_ref[:], preferred_element_type=jnp.float32, precision=lax.Precision.HIGHEST)
       * inv_ref[:]
       + jnp.dot(h_ref[:], rw_ref[:], preferred_element_type=jnp.float32, precision=lax.Precision.HIGHEST)
       + b_ref[:])
  out_ref[:] = jnp.maximum(v, 0.0) if relu else v


_row_specs = [
    pl.BlockSpec((BN, 256), lambda i: (i, 0)),
    pl.BlockSpec((256, H), lambda i: (0, 0)),
    pl.BlockSpec((BN, H), lambda i: (i, 0)),
    pl.BlockSpec((H, H), lambda i: (0, 0)),
    pl.BlockSpec((1, H), lambda i: (0, 0)),
    pl.BlockSpec((BN, H), lambda i: (i, 0)),
]
_row_out = pl.BlockSpec((BN, H), lambda i: (i, 0))


def _layer1(g2, wm, h, rw, b2, cnt):
  return pl.pallas_call(
      _layer1_body,
      grid=(N // BN,),
      in_specs=_row_specs,
      out_specs=[_row_out, _row_out],
      out_shape=[jax.ShapeDtypeStruct((N, H), jnp.float32),
                 jax.ShapeDtypeStruct((N, H), jnp.float32)],
  )(g2, wm, h, rw, b2, cnt)


def _layerk(g2, wm, h, rw, b2, inv, relu):
  return pl.pallas_call(
      functools.partial(_layerk_body, relu),
      grid=(N // BN,),
      in_specs=_row_specs,
      out_specs=_row_out,
      out_shape=jax.ShapeDtypeStruct((N, H), jnp.float32),
  )(g2, wm, h, rw, b2, inv)


def _readout_body(h_hbm, sids_hbm, tids_hbm, xmid_hbm, mmid_hbm, fw_hbm,
                  fb_hbm, res_hbm, sid_v, tid_v, xmid_v, mmid_v, xm_v, mm_v,
                  xs_v, xt_v, fw_v, fb_v, out_v, sem):
  c = lax.axis_index("c")
  s = lax.axis_index("s")

  @pl.when((c == 0) & (s == 0))
  def _():
    pltpu.sync_copy(sids_hbm, sid_v)
    pltpu.sync_copy(tids_hbm, tid_v)
    pltpu.sync_copy(xmid_hbm, xmid_v)
    pltpu.sync_copy(mmid_hbm, mmid_v)
    pltpu.sync_copy(fw_hbm, fw_v)
    pltpu.sync_copy(fb_hbm, fb_v)
    pltpu.async_copy(h_hbm.at[xmid_v], xm_v.at[pl.ds(8, 8)], sem).wait()
    pltpu.async_copy(h_hbm.at[mmid_v], mm_v, sem).wait()
    lane = lax.iota(jnp.int32, 16)
    for b in range(B):
      pltpu.async_copy(h_hbm.at[sid_v.at[b]], xs_v, sem).wait()
      pltpu.async_copy(h_hbm.at[tid_v.at[b]], xt_v, sem).wait()
      fhm = jnp.sum(mm_v[b] * fw_v[...])
      acc = jnp.zeros((16,), jnp.float32)
      bb = jnp.full((16,), b + 8, jnp.int32)
      for i in range(H):
        ii = jnp.full((16,), i, jnp.int32)
        xs_i = plsc.load_gather(xs_v, [lane, ii])
        xt_i = plsc.load_gather(xt_v, [lane, ii])
        xm_i = plsc.load_gather(xm_v, [bb, ii])
        acc = acc + xm_i * (xs_i + xt_i) - xs_i * xt_i
      res = jnp.where(lane < 8, acc - fhm - fb_v[...], -acc)
      out_v[b, :] = res
    pltpu.sync_copy(out_v, res_hbm)


_readout = pl.kernel(
    _readout_body,
    out_type=jax.ShapeDtypeStruct((B, 16), jnp.float32),
    mesh=_mesh,
    scratch_types=[
        pltpu.VMEM((B, 16), jnp.int32),
        pltpu.VMEM((B, 16), jnp.int32),
        pltpu.VMEM((B,), jnp.int32),
        pltpu.VMEM((B,), jnp.int32),
        pltpu.VMEM((16, H), jnp.float32),
        pltpu.VMEM((B, H), jnp.float32),
        pltpu.VMEM((16, H), jnp.float32),
        pltpu.VMEM((16, H), jnp.float32),
        pltpu.VMEM((16,), jnp.float32),
        pltpu.VMEM((16,), jnp.float32),
        pltpu.VMEM((B, 16), jnp.float32),
        pltpu.SemaphoreType.DMA,
    ],
    compiler_params=pltpu.CompilerParams(use_tc_tiling_on_sc=False,
                                         needs_layout_passes=False))


def kernel(x, edge_index, edge_attr, batch, loop_edge, loop_pair, metal_idx,
           node_emb,
           c1_edge_emb, c1_nn_w, c1_nn_b, c1_root_w, c1_bias,
           c2_edge_emb, c2_nn_w, c2_nn_b, c2_root_w, c2_bias,
           c3_edge_emb, c3_nn_w, c3_nn_b, c3_root_w, c3_bias,
           c4_edge_emb, c4_nn_w, c4_nn_b, c4_root_w, c4_bias,
           f_w, f_b):
  src = edge_index[0].astype(jnp.int32)
  dst = edge_index[1].astype(jnp.int32)
  attr = edge_attr.astype(jnp.int32)
  zeros = jnp.zeros((ZPT, H), jnp.float32)
  ones = jnp.ones((CH, H), jnp.float32)

  estack = jnp.stack([c1_edge_emb, c2_edge_emb, c3_edge_emb, c4_edge_emb])
  wstack = jnp.stack([c1_nn_w, c2_nn_w, c3_nn_w, c4_nn_w])
  bstack = jnp.stack([c1_nn_b, c2_nn_b, c3_nn_b, c4_nn_b]).reshape(4, 1, 256)
  wflat = _wgen(estack, wstack, bstack)
  wms = jnp.reshape(wflat, (4, 256, H))

  rws = [c1_root_w, c2_root_w, c3_root_w, c4_root_w]
  bs = [b.reshape(1, H) for b in (c1_bias, c2_bias, c3_bias, c4_bias)]

  emb_pad = jnp.zeros((128, H), jnp.float32).at[:node_emb.shape[0]].set(
      node_emb.astype(jnp.float32))
  h = _h0(x.astype(jnp.int32).reshape(N, 1), emb_pad)

  g, cnt = _scatter_cnt(h, src, dst, attr, zeros, ones)
  h, inv = _layer1(jnp.reshape(g, (N, 256)), wms[0], h, rws[0], bs[0], cnt)
  for l in (1, 2, 3):
    g = _scatter(h, src, dst, attr, zeros, ones)
    h = _layerk(jnp.reshape(g, (N, 256)), wms[l], h, rws[l], bs[l], inv,
                relu=(l < 3))

  bvec = (jnp.arange(B, dtype=jnp.int32) * NPER)[:, None]
  le = loop_edge.astype(jnp.int32)
  lp = loop_pair.astype(jnp.int32)
  sids = jnp.concatenate([bvec + le[..., 0], bvec + lp[..., 0]], axis=1)
  tids = jnp.concatenate([bvec + le[..., 1], bvec + lp[..., 1]], axis=1)
  xmid = bvec[:, 0] + metal_idx.astype(jnp.int32)
  mmid = metal_idx.astype(jnp.int32)
  fw16 = f_w.astype(jnp.float32).reshape(H)
  fb16 = jnp.broadcast_to(f_b.astype(jnp.float32), (16,))
  return _readout(h, sids, tids, xmid, mmid, fw16, fb16)


# speedup vs baseline: 11.1262x; 3.1966x over previous
import functools

import jax
import jax.numpy as jnp
from jax import lax
from jax.experimental import pallas as pl
from jax.experimental.pallas import tpu as pltpu
from jax.experimental.pallas import tpu_sc as plsc

N = 10000
E = 160000
H = 16
B = 8
NPER = N // B
NC = 2
NS = 16
HALF_N = N // NC
HALF_ROWS = HALF_N * 16
G_ROWS = HALF_ROWS + 8
ZPT = 5008
CNT_ROWS = HALF_N + 8
CH = 80
EPT = E // NS
NCHUNK = EPT // CH
NBUF = 5

_mesh = plsc.VectorSubcoreMesh(
    core_axis_name="c", subcore_axis_name="s", num_cores=NC, num_subcores=NS)


def _scatter_body(h_hbm, src_hbm, dst_hbm, attr_hbm, zeros_hbm, g_hbm,
                  src_v, dst_v, attr_v, key_v, g_sp, rows, gsems, ssems):
  c = lax.axis_index("c")
  s = lax.axis_index("s")

  off = pl.multiple_of(s * EPT, 8)
  pltpu.sync_copy(src_hbm.at[pl.ds(off, EPT)], src_v)
  pltpu.sync_copy(dst_hbm.at[pl.ds(off, EPT)], dst_v)
  pltpu.sync_copy(attr_hbm.at[pl.ds(off, EPT)], attr_v)
  zoff = jnp.where(s == NS - 1, G_ROWS - ZPT, s * (HALF_ROWS // NS))
  pltpu.sync_copy(zeros_hbm, g_sp.at[pl.ds(pl.multiple_of(zoff, 8), ZPT)])

  def keys(j, carry):
    sl = pl.ds(j * 16, 16)
    d = dst_v[sl]
    a = attr_v[sl]
    k = d * 16 + a - c * HALF_ROWS
    ok = (k >= 0) & (k < HALF_ROWS)
    key_v[sl] = jnp.where(ok, k, HALF_ROWS + (d & 7))
    return carry

  lax.fori_loop(0, EPT // 16, keys, 0)
  plsc.subcore_barrier()

  def group(g, carry):
    gds = []
    for i in range(NBUF):
      t = g * NBUF + i
      gds.append(pltpu.async_copy(
          h_hbm.at[src_v.at[pl.ds(t * CH, CH)]], rows[i], gsems[i]))
    sds = []
    for i in range(NBUF):
      t = g * NBUF + i
      gds[i].wait()
      sds.append(pltpu.async_copy(
          rows[i], g_sp.at[key_v.at[pl.ds(t * CH, CH)]], ssems[i],
          add=True))
    for d in sds:
      d.wait()
    return carry

  lax.fori_loop(0, NCHUNK // NBUF, group, 0)
  plsc.subcore_barrier()

  dpt = HALF_ROWS // NS
  pltpu.sync_copy(g_sp.at[pl.ds(s * dpt, dpt)],
                  g_hbm.at[pl.ds(c * HALF_ROWS + s * dpt, dpt)])


_scatter = pl.kernel(
    _scatter_body,
    out_type=jax.ShapeDtypeStruct((N * 16, H), jnp.float32),
    mesh=_mesh,
    scratch_types=[
        pltpu.VMEM((EPT,), jnp.int32),
        pltpu.VMEM((EPT,), jnp.int32),
        pltpu.VMEM((EPT,), jnp.int32),
        pltpu.VMEM((EPT,), jnp.int32),
        pltpu.VMEM_SHARED((G_ROWS, H), jnp.float32),
        [pltpu.VMEM((CH, H), jnp.float32) for _ in range(NBUF)],
        [pltpu.SemaphoreType.DMA for _ in range(NBUF)],
        [pltpu.SemaphoreType.DMA for _ in range(NBUF)],
    ],
    compiler_params=pltpu.CompilerParams(use_tc_tiling_on_sc=False))


def _scatter_cnt_body(h_hbm, src_hbm, dst_hbm, attr_hbm, zeros_hbm, ones_hbm,
                      g_hbm, cnt_hbm,
                      src_v, dst_v, attr_v, ones_v,
                      g_sp, cnt_sp, rows, gsems, ssems, csems):
  c = lax.axis_index("c")
  s = lax.axis_index("s")

  off = pl.multiple_of(s * EPT, 8)
  pltpu.sync_copy(src_hbm.at[pl.ds(off, EPT)], src_v)
  pltpu.sync_copy(dst_hbm.at[pl.ds(off, EPT)], dst_v)
  pltpu.sync_copy(attr_hbm.at[pl.ds(off, EPT)], attr_v)
  pltpu.sync_copy(ones_hbm, ones_v)
  zoff = jnp.where(s == NS - 1, G_ROWS - ZPT, s * (HALF_ROWS // NS))
  pltpu.sync_copy(zeros_hbm, g_sp.at[pl.ds(pl.multiple_of(zoff, 8), ZPT)])
  czoff = jnp.where(s == NS - 1, CNT_ROWS - 320, s * 312)
  pltpu.sync_copy(zeros_hbm.at[pl.ds(0, 320)],
                  cnt_sp.at[pl.ds(pl.multiple_of(czoff, 8), 320)])

  def keys(j, carry):
    sl = pl.ds(j * 16, 16)
    d = dst_v[sl]
    a = attr_v[sl]
    k = d * 16 + a - c * HALF_ROWS
    ok = (k >= 0) & (k < HALF_ROWS)
    dst_v[sl] = jnp.where(ok, k, HALF_ROWS + (d & 7))
    dl = d - c * HALF_N
    okd = (dl >= 0) & (dl < HALF_N)
    attr_v[sl] = jnp.where(okd, dl, HALF_N + (d & 7))
    return carry

  lax.fori_loop(0, EPT // 16, keys, 0)
  plsc.subcore_barrier()

  def group(g, carry):
    gds = []
    for i in range(NBUF):
      t = g * NBUF + i
      gds.append(pltpu.async_copy(
          h_hbm.at[src_v.at[pl.ds(t * CH, CH)]], rows[i], gsems[i]))
    sds = []
    for i in range(NBUF):
      t = g * NBUF + i
      sds.append(pltpu.async_copy(
          ones_v, cnt_sp.at[attr_v.at[pl.ds(t * CH, CH)]], csems[i],
          add=True))
      gds[i].wait()
      sds.append(pltpu.async_copy(
          rows[i], g_sp.at[dst_v.at[pl.ds(t * CH, CH)]], ssems[i],
          add=True))
    for d in sds:
      d.wait()
    return carry

  lax.fori_loop(0, NCHUNK // NBUF, group, 0)
  plsc.subcore_barrier()

  dpt = HALF_ROWS // NS
  pltpu.sync_copy(g_sp.at[pl.ds(s * dpt, dpt)],
                  g_hbm.at[pl.ds(c * HALF_ROWS + s * dpt, dpt)])

  @pl.when(s == 0)
  def _():
    pltpu.sync_copy(cnt_sp.at[pl.ds(0, HALF_N)],
                    cnt_hbm.at[pl.ds(c * HALF_N, HALF_N)])


_scatter_cnt = pl.kernel(
    _scatter_cnt_body,
    out_type=[jax.ShapeDtypeStruct((N * 16, H), jnp.float32),
              jax.ShapeDtypeStruct((N, H), jnp.float32)],
    mesh=_mesh,
    scratch_types=[
        pltpu.VMEM((EPT,), jnp.int32),
        pltpu.VMEM((EPT,), jnp.int32),
        pltpu.VMEM((EPT,), jnp.int32),
        pltpu.VMEM((CH, H), jnp.float32),
        pltpu.VMEM_SHARED((G_ROWS, H), jnp.float32),
        pltpu.VMEM_SHARED((CNT_ROWS, H), jnp.float32),
        [pltpu.VMEM((CH, H), jnp.float32) for _ in range(NBUF)],
        [pltpu.SemaphoreType.DMA for _ in range(NBUF)],
        [pltpu.SemaphoreType.DMA for _ in range(NBUF)],
        [pltpu.SemaphoreType.DMA for _ in range(NBUF)],
    ],
    compiler_params=pltpu.CompilerParams(use_tc_tiling_on_sc=False))


def _wgen_body(e_ref, w_ref, b_ref, out_ref):
  for l in range(4):
    e = jax.nn.relu(e_ref[l])
    out_ref[l] = jnp.dot(e, w_ref[l],
                         preferred_element_type=jnp.float32, precision=lax.Precision.HIGHEST) + b_ref[l]


def _wgen(estack, wstack, bstack):
  return pl.pallas_call(
      _wgen_body,
      out_shape=jax.ShapeDtypeStruct((4, 16, 256), jnp.float32),
  )(estack, wstack, bstack)


BN0 = 2000


def _h0_body(x_ref, emb_ref, out_ref):
  iota = lax.broadcasted_iota(jnp.int32, (BN0, 128), 1)
  oh = (iota == x_ref[:]).astype(jnp.float32)
  out_ref[:] = jnp.dot(oh, emb_ref[:], preferred_element_type=jnp.float32, precision=lax.Precision.HIGHEST)


def _h0(x2, emb_pad):
  return pl.pallas_call(
      _h0_body,
      grid=(N // BN0,),
      in_specs=[
          pl.BlockSpec((BN0, 1), lambda i: (i, 0)),
          pl.BlockSpec((128, H), lambda i: (0, 0)),
      ],
      out_specs=pl.BlockSpec((BN0, H), lambda i: (i, 0)),
      out_shape=jax.ShapeDtypeStruct((N, H), jnp.float32),
  )(x2, emb_pad)


BN = 2000


def _layer1_body(g_ref, wm_ref, h_ref, rw_ref, b_ref, cnt_ref,
                 out_ref, inv_ref):
  inv = 1.0 / jnp.maximum(cnt_ref[:], 1.0)
  inv_ref[:] = inv
  v = (jnp.dot(g_ref[:], wm_ref[:], preferred_element_type=jnp.float32, precision=lax.Precision.HIGHEST) * inv
       + jnp.dot(h_ref[:], rw_ref[:], preferred_element_type=jnp.float32, precision=lax.Precision.HIGHEST)
       + b_ref[:])
  out_ref[:] = jnp.maximum(v, 0.0)


def _layerk_body(relu, g_ref, wm_ref, h_ref, rw_ref, b_ref, inv_ref, out_ref):
  v = (jnp.dot(g_ref[:], wm_ref[:], preferred_element_type=jnp.float32, precision=lax.Precision.HIGHEST)
       * inv_ref[:]
       + jnp.dot(h_ref[:], rw_ref[:], preferred_element_type=jnp.float32, precision=lax.Precision.HIGHEST)
       + b_ref[:])
  out_ref[:] = jnp.maximum(v, 0.0) if relu else v


_row_specs = [
    pl.BlockSpec((BN, 256), lambda i: (i, 0)),
    pl.BlockSpec((256, H), lambda i: (0, 0)),
    pl.BlockSpec((BN, H), lambda i: (i, 0)),
    pl.BlockSpec((H, H), lambda i: (0, 0)),
    pl.BlockSpec((1, H), lambda i: (0, 0)),
    pl.BlockSpec((BN, H), lambda i: (i, 0)),
]
_row_out = pl.BlockSpec((BN, H), lambda i: (i, 0))


def _layer1(g2, wm, h, rw, b2, cnt):
  return pl.pallas_call(
      _layer1_body,
      grid=(N // BN,),
      in_specs=_row_specs,
      out_specs=[_row_out, _row_out],
      out_shape=[jax.ShapeDtypeStruct((N, H), jnp.float32),
                 jax.ShapeDtypeStruct((N, H), jnp.float32)],
  )(g2, wm, h, rw, b2, cnt)


def _layerk(g2, wm, h, rw, b2, inv, relu):
  return pl.pallas_call(
      functools.partial(_layerk_body, relu),
      grid=(N // BN,),
      in_specs=_row_specs,
      out_specs=_row_out,
      out_shape=jax.ShapeDtypeStruct((N, H), jnp.float32),
  )(g2, wm, h, rw, b2, inv)


def _readout_body(h_hbm, sids_hbm, tids_hbm, xmid_hbm, mmid_hbm, fw_hbm,
                  fb_hbm, res_hbm, sid_v, tid_v, xmid_v, mmid_v, xm_v, mm_v,
                  xs_v, xt_v, fw_v, fb_v, out_v, sem):
  c = lax.axis_index("c")
  s = lax.axis_index("s")

  @pl.when((c == 0) & (s == 0))
  def _():
    pltpu.sync_copy(sids_hbm, sid_v)
    pltpu.sync_copy(tids_hbm, tid_v)
    pltpu.sync_copy(xmid_hbm, xmid_v)
    pltpu.sync_copy(mmid_hbm, mmid_v)
    pltpu.sync_copy(fw_hbm, fw_v)
    pltpu.sync_copy(fb_hbm, fb_v)
    pltpu.async_copy(h_hbm.at[xmid_v], xm_v.at[pl.ds(8, 8)], sem).wait()
    pltpu.async_copy(h_hbm.at[mmid_v], mm_v, sem).wait()
    lane = lax.iota(jnp.int32, 16)
    for b in range(B):
      pltpu.async_copy(h_hbm.at[sid_v.at[b]], xs_v, sem).wait()
      pltpu.async_copy(h_hbm.at[tid_v.at[b]], xt_v, sem).wait()
      fhm = jnp.sum(mm_v[b] * fw_v[...])
      acc = jnp.zeros((16,), jnp.float32)
      bb = jnp.full((16,), b + 8, jnp.int32)
      for i in range(H):
        ii = jnp.full((16,), i, jnp.int32)
        xs_i = plsc.load_gather(xs_v, [lane, ii])
        xt_i = plsc.load_gather(xt_v, [lane, ii])
        xm_i = plsc.load_gather(xm_v, [bb, ii])
        acc = acc + xm_i * (xs_i + xt_i) - xs_i * xt_i
      res = jnp.where(lane < 8, acc - fhm - fb_v[...], -acc)
      out_v[b, :] = res
    pltpu.sync_copy(out_v, res_hbm)


_readout = pl.kernel(
    _readout_body,
    out_type=jax.ShapeDtypeStruct((B, 16), jnp.float32),
    mesh=_mesh,
    scratch_types=[
        pltpu.VMEM((B, 16), jnp.int32),
        pltpu.VMEM((B, 16), jnp.int32),
        pltpu.VMEM((B,), jnp.int32),
        pltpu.VMEM((B,), jnp.int32),
        pltpu.VMEM((16, H), jnp.float32),
        pltpu.VMEM((B, H), jnp.float32),
        pltpu.VMEM((16, H), jnp.float32),
        pltpu.VMEM((16, H), jnp.float32),
        pltpu.VMEM((16,), jnp.float32),
        pltpu.VMEM((16,), jnp.float32),
        pltpu.VMEM((B, 16), jnp.float32),
        pltpu.SemaphoreType.DMA,
    ],
    compiler_params=pltpu.CompilerParams(use_tc_tiling_on_sc=False,
                                         needs_layout_passes=False))


def kernel(x, edge_index, edge_attr, batch, loop_edge, loop_pair, metal_idx,
           node_emb,
           c1_edge_emb, c1_nn_w, c1_nn_b, c1_root_w, c1_bias,
           c2_edge_emb, c2_nn_w, c2_nn_b, c2_root_w, c2_bias,
           c3_edge_emb, c3_nn_w, c3_nn_b, c3_root_w, c3_bias,
           c4_edge_emb, c4_nn_w, c4_nn_b, c4_root_w, c4_bias,
           f_w, f_b):
  src = edge_index[0].astype(jnp.int32)
  dst = edge_index[1].astype(jnp.int32)
  attr = edge_attr.astype(jnp.int32)
  zeros = jnp.zeros((ZPT, H), jnp.float32)
  ones = jnp.ones((CH, H), jnp.float32)

  estack = jnp.stack([c1_edge_emb, c2_edge_emb, c3_edge_emb, c4_edge_emb])
  wstack = jnp.stack([c1_nn_w, c2_nn_w, c3_nn_w, c4_nn_w])
  bstack = jnp.stack([c1_nn_b, c2_nn_b, c3_nn_b, c4_nn_b]).reshape(4, 1, 256)
  wflat = _wgen(estack, wstack, bstack)
  wms = jnp.reshape(wflat, (4, 256, H))

  rws = [c1_root_w, c2_root_w, c3_root_w, c4_root_w]
  bs = [b.reshape(1, H) for b in (c1_bias, c2_bias, c3_bias, c4_bias)]

  emb_pad = jnp.zeros((128, H), jnp.float32).at[:node_emb.shape[0]].set(
      node_emb.astype(jnp.float32))
  h = _h0(x.astype(jnp.int32).reshape(N, 1), emb_pad)

  g, cnt = _scatter_cnt(h, src, dst, attr, zeros, ones)
  h, inv = _layer1(jnp.reshape(g, (N, 256)), wms[0], h, rws[0], bs[0], cnt)
  for l in (1, 2, 3):
    g = _scatter(h, src, dst, attr, zeros)
    h = _layerk(jnp.reshape(g, (N, 256)), wms[l], h, rws[l], bs[l], inv,
                relu=(l < 3))

  bvec = (jnp.arange(B, dtype=jnp.int32) * NPER)[:, None]
  le = loop_edge.astype(jnp.int32)
  lp = loop_pair.astype(jnp.int32)
  sids = jnp.concatenate([bvec + le[..., 0], bvec + lp[..., 0]], axis=1)
  tids = jnp.concatenate([bvec + le[..., 1], bvec + lp[..., 1]], axis=1)
  xmid = bvec[:, 0] + metal_idx.astype(jnp.int32)
  mmid = metal_idx.astype(jnp.int32)
  fw16 = f_w.astype(jnp.float32).reshape(H)
  fb16 = jnp.broadcast_to(f_b.astype(jnp.float32), (16,))
  return _readout(h, sids, tids, xmid, mmid, fw16, fb16)
